# Initial kernel scaffold; baseline (speedup 1.0000x reference)
import jax
import jax.numpy as jnp
from jax.experimental import pallas as pl


def kernel(x, edge_index, batch, W1, b1, g1, bt1, W2, b2, g2, bt2, W3, b3, g3, bt3, V1, c1, V2, c2):
    raise NotImplementedError("write your pallas kernel here")


# speedup vs baseline: 3.0599x; 3.0599x over previous
import functools

import jax
import jax.numpy as jnp
from jax import lax
from jax.experimental import pallas as pl
from jax.experimental.pallas import tpu as pltpu
from jax.experimental.pallas import tpu_sc as plsc

_NC, _NS = 2, 16
_NW = _NC * _NS
_IW = 100
_IR = 10
_CE = _IW * _IR
_G = 64
_NP = 10240
_BN = 256
_BE = 2000


def _node_mm_body(x_ref, wa_ref, wb_ref, b1_ref, ab_ref):
    xb = x_ref[...]
    a = jnp.dot(xb, wa_ref[...], preferred_element_type=jnp.float32)
    b = jnp.dot(xb, wb_ref[...], preferred_element_type=jnp.float32)
    ab_ref[...] = jnp.concatenate([a + b1_ref[...], b], axis=1)


def _gather_body(ab_hbm, src_hbm, dst_hbm, z_hbm,
                 idxs_v, idxd_v, ga, gb, zb,
                 sa0, sa1, sb0, sb1, sz0, sz1):
    cid = lax.axis_index("c")
    sid = lax.axis_index("s")
    wid = sid * _NC + cid
    chunks_per_w = src_hbm.shape[0] // _NW
    c0 = wid * chunks_per_w
    sas, sbs, szs = (sa0, sa1), (sb0, sb1), (sz0, sz1)

    def chunk(i, carry):
        ci = c0 + i
        pltpu.sync_copy(dst_hbm.at[ci], idxd_v)
        pltpu.sync_copy(src_hbm.at[ci], idxs_v)

        def start(s):
            k = s % 2
            return (
                pltpu.async_copy(ab_hbm.at[idxd_v.at[s]], ga.at[k], sas[k]),
                pltpu.async_copy(ab_hbm.at[idxs_v.at[s]], gb.at[k], sbs[k]),
            )

        cps = {0: start(0)}
        zcps = {}
        for s in range(_IR):
            if s + 1 < _IR:
                cps[s + 1] = start(s + 1)
            ca, cb = cps.pop(s)
            ca.wait()
            cb.wait()
            if s - 2 in zcps:
                zcps.pop(s - 2).wait()
            k = s % 2

            def fold(r, c2, k=k):
                for q in range(4):
                    sl = pl.ds(q * 16, 16)
                    slb = pl.ds(64 + q * 16, 16)
                    zb[k, r, sl] = ga[k, r, sl] + gb[k, r, slb]
                return c2

            lax.fori_loop(0, _IW, fold, 0)
            zcps[s] = pltpu.async_copy(zb.at[k], z_hbm.at[ci * _IR + s], szs[k])
        for s in (_IR - 2, _IR - 1):
            zcps.pop(s).wait()
        return carry

    lax.fori_loop(0, chunks_per_w, chunk, 0)


def _stats_body(z_ref, s_ref):
    i = pl.program_id(0)

    @pl.when(i == 0)
    def _init():
        s_ref[...] = jnp.zeros_like(s_ref)

    zb = z_ref[...]
    hh = zb.shape[2]
    s0 = jnp.sum(zb, axis=(0, 1)).reshape(1, hh)
    s1 = jnp.sum(zb * zb, axis=(0, 1)).reshape(1, hh)
    pad = jnp.zeros((6, hh), jnp.float32)
    s_ref[...] = s_ref[...] + jnp.concatenate([s0, s1, pad], axis=0)


def _mid_body(stats_ref, z_ref, w_ref, bv_ref, g_ref, bt_ref,
              z2_ref, s2_ref, *, ne):
    i = pl.program_id(0)
    hh = w_ref.shape[0]
    m = stats_ref[0:1, :] / ne
    v = stats_ref[1:2, :] / ne - m * m
    al = g_ref[...] * lax.rsqrt(v + 1e-5)
    be = bt_ref[...] - m * al
    h = jnp.maximum(z_ref[...] * al.reshape(1, 1, hh) + be.reshape(1, 1, hh),
                    0.0)
    z2 = lax.dot_general(h, w_ref[...], (((2,), (0,)), ((), ())),
                         preferred_element_type=jnp.float32)
    z2 = z2 + bv_ref[...].reshape(1, 1, hh)
    z2_ref[...] = z2

    @pl.when(i == 0)
    def _init():
        s2_ref[...] = jnp.zeros_like(s2_ref)

    s0 = jnp.sum(z2, axis=(0, 1)).reshape(1, hh)
    s1 = jnp.sum(z2 * z2, axis=(0, 1)).reshape(1, hh)
    pad = jnp.zeros((6, hh), jnp.float32)
    s2_ref[...] = s2_ref[...] + jnp.concatenate([s0, s1, pad], axis=0)


def _act_body(stats_ref, z_ref, g_ref, bt_ref, h_ref, *, ne):
    hh = z_ref.shape[2]
    m = stats_ref[0:1, :] / ne
    v = stats_ref[1:2, :] / ne - m * m
    al = g_ref[...] * lax.rsqrt(v + 1e-5)
    be = bt_ref[...] - m * al
    h = jnp.maximum(z_ref[...] * al.reshape(1, 1, hh) + be.reshape(1, 1, hh),
                    0.0)
    zb, iw = h.shape[0], h.shape[1]
    h_ref[...] = jnp.concatenate(
        [h, jnp.ones((zb, iw, 1), jnp.float32),
         jnp.zeros((zb, iw, 63), jnp.float32)], axis=2)


_SR = 2
_SE = _SR * _IW


def _scatter_body(h_hbm, dst_hbm, zz_hbm, o_hbm, idx_v, vals_v, acc_sh, semv):
    cid = lax.axis_index("c")
    sid = lax.axis_index("s")
    wid = sid * _NC + cid
    stripe = _NP // _NS
    st = sid * stripe
    pltpu.sync_copy(zz_hbm.at[pl.ds(st, stripe)], acc_sh.at[pl.ds(st, stripe)])
    plsc.subcore_barrier()

    chunks_per_w = dst_hbm.shape[0] // _NW
    c0 = wid * chunks_per_w

    def chunk(i, c):
        ci = c0 + i
        pltpu.sync_copy(dst_hbm.at[ci], idx_v)
        pltpu.async_copy(h_hbm.at[pl.ds(_SR * ci, _SR)], vals_v, semv).wait()
        for s in range(_SR):
            pltpu.sync_copy(vals_v.at[s], acc_sh.at[idx_v.at[s]], add=True)
        return c

    lax.fori_loop(0, chunks_per_w, chunk, 0)
    plsc.subcore_barrier()
    base = cid * _NP + st
    pltpu.sync_copy(acc_sh.at[pl.ds(st, stripe)], o_hbm.at[pl.ds(base, stripe)])


def _final_body(p0_ref, p1_ref, x_ref, b_ref,
                v1_ref, c1_ref, v2_ref, c2_ref, y_ref, acc_ref,
                *, nn, nblocks):
    i = pl.program_id(0)

    @pl.when(i == 0)
    def _init():
        acc_ref[...] = jnp.zeros_like(acc_ref)

    aggs = p0_ref[...] + p1_ref[...]
    deg = aggs[:, 64:65]
    agg = aggs[:, :64] / jnp.maximum(deg, 1.0)
    xb = x_ref[...]
    rows = lax.broadcasted_iota(jnp.int32, (_BN, 1), 0) + i * _BN
    valid = (rows < nn).astype(jnp.float32)
    outb = jnp.concatenate(
        [agg, xb, valid, jnp.zeros((_BN, 63), jnp.float32)], axis=1)
    outb = jnp.where(valid > 0.0, outb, 0.0)
    bvec = b_ref[0]
    oh = (bvec == lax.broadcasted_iota(jnp.int32, (_G, _BN), 0))
    oh = oh.astype(jnp.float32)
    acc_ref[...] = acc_ref[...] + jnp.dot(
        oh, outb, preferred_element_type=jnp.float32)

    @pl.when(i == nblocks - 1)
    def _fin():
        ps = acc_ref[...]
        pooled = ps[:, :320] / jnp.maximum(ps[:, 320:321], 1.0)
        t = jnp.dot(pooled, v1_ref[...], preferred_element_type=jnp.float32)
        t = jnp.maximum(t + c1_ref[...], 0.0)
        yy = jnp.dot(t, v2_ref[...], preferred_element_type=jnp.float32)
        yy = yy + c2_ref[...]
        y_ref[...] = 1.0 / (1.0 + jnp.exp(-yy))


def kernel(x, edge_index, batch,
           W1, b1, g1, bt1, W2, b2, g2, bt2, W3, b3, g3, bt3,
           V1, c1, V2, c2):
    n, d = x.shape
    e = edge_index.shape[1]
    h = W1.shape[1]
    ne = float(e)
    src3 = edge_index[0].reshape(e // _CE, _IR, _IW)
    dst3 = edge_index[1].reshape(e // _CE, _IR, _IW)
    wa = W1[:d] - W1[d:]
    wb = W1[d:]
    b1r = b1.reshape(1, h)

    nb_n = (n + _BN - 1) // _BN
    ab = pl.pallas_call(
        _node_mm_body,
        grid=(nb_n,),
        in_specs=[
            pl.BlockSpec((_BN, d), lambda i: (i, 0)),
            pl.BlockSpec((d, h), lambda i: (0, 0)),
            pl.BlockSpec((d, h), lambda i: (0, 0)),
            pl.BlockSpec((1, h), lambda i: (0, 0)),
        ],
        out_specs=pl.BlockSpec((_BN, 2 * h), lambda i: (i, 0)),
        out_shape=jax.ShapeDtypeStruct((n, 2 * h), jnp.float32),
    )(x, wa, wb, b1r)

    mesh = plsc.VectorSubcoreMesh(core_axis_name="c", subcore_axis_name="s")
    nslab = e // _IW
    z1 = pl.kernel(
        _gather_body,
        out_type=jax.ShapeDtypeStruct((nslab, _IW, h), jnp.float32),
        mesh=mesh,
        scratch_types=[
            pltpu.VMEM((_IR, _IW), jnp.int32),
            pltpu.VMEM((_IR, _IW), jnp.int32),
            pltpu.VMEM((2, _IW, 2 * h), jnp.float32),
            pltpu.VMEM((2, _IW, 2 * h), jnp.float32),
            pltpu.VMEM((2, _IW, h), jnp.float32),
            pltpu.SemaphoreType.DMA,
            pltpu.SemaphoreType.DMA,
            pltpu.SemaphoreType.DMA,
            pltpu.SemaphoreType.DMA,
            pltpu.SemaphoreType.DMA,
            pltpu.SemaphoreType.DMA,
        ],
    )(ab, src3, dst3)

    _ZB = _BE // _IW
    nb_e = nslab // _ZB
    s1 = pl.pallas_call(
        _stats_body,
        grid=(nb_e,),
        in_specs=[pl.BlockSpec((_ZB, _IW, h), lambda i: (i, 0, 0))],
        out_specs=pl.BlockSpec((8, h), lambda i: (0, 0)),
        out_shape=jax.ShapeDtypeStruct((8, h), jnp.float32),
    )(z1)

    def mid(stats, z, w, bv, g, bt):
        return pl.pallas_call(
            functools.partial(_mid_body, ne=ne),
            grid=(nb_e,),
            in_specs=[
                pl.BlockSpec((8, h), lambda i: (0, 0)),
                pl.BlockSpec((_ZB, _IW, h), lambda i: (i, 0, 0)),
                pl.BlockSpec((h, h), lambda i: (0, 0)),
                pl.BlockSpec((1, h), lambda i: (0, 0)),
                pl.BlockSpec((1, h), lambda i: (0, 0)),
                pl.BlockSpec((1, h), lambda i: (0, 0)),
            ],
            out_specs=[
                pl.BlockSpec((_ZB, _IW, h), lambda i: (i, 0, 0)),
                pl.BlockSpec((8, h), lambda i: (0, 0)),
            ],
            out_shape=[
                jax.ShapeDtypeStruct((nslab, _IW, h), jnp.float32),
                jax.ShapeDtypeStruct((8, h), jnp.float32),
            ],
        )(stats, z, w, bv.reshape(1, h), g.reshape(1, h), bt.reshape(1, h))

    z2, s2 = mid(s1, z1, W2, b2, g1, bt1)
    z3, s3 = mid(s2, z2, W3, b3, g2, bt2)

    h3 = pl.pallas_call(
        functools.partial(_act_body, ne=ne),
        grid=(nb_e,),
        in_specs=[
            pl.BlockSpec((8, h), lambda i: (0, 0)),
            pl.BlockSpec((_ZB, _IW, h), lambda i: (i, 0, 0)),
            pl.BlockSpec((1, h), lambda i: (0, 0)),
            pl.BlockSpec((1, h), lambda i: (0, 0)),
        ],
        out_specs=pl.BlockSpec((_ZB, _IW, 2 * h), lambda i: (i, 0, 0)),
        out_shape=jax.ShapeDtypeStruct((nslab, _IW, 2 * h), jnp.float32),
    )(s3, z3, g3.reshape(1, h), bt3.reshape(1, h))

    zz = jnp.zeros((_NP, 2 * h), jnp.float32)
    dst4 = edge_index[1].reshape(e // _SE, _SR, _IW)
    o = pl.kernel(
        _scatter_body,
        out_type=jax.ShapeDtypeStruct((2 * _NP, 2 * h), jnp.float32),
        mesh=mesh,
        scratch_types=[
            pltpu.VMEM((_SR, _IW), jnp.int32),
            pltpu.VMEM((_SR, _IW, 2 * h), jnp.float32),
            pltpu.VMEM_SHARED((_NP, 2 * h), jnp.float32),
            pltpu.SemaphoreType.DMA,
        ],
    )(h3, dst4, zz)

    nb_p = _NP // _BN
    batch_p = jnp.pad(batch, (0, _NP - n), constant_values=_G)
    batch_p = batch_p.reshape(nb_p, 1, _BN)
    y = pl.pallas_call(
        functools.partial(_final_body, nn=n, nblocks=nb_p),
        grid=(nb_p,),
        in_specs=[
            pl.BlockSpec((_BN, 2 * h), lambda i: (i, 0)),
            pl.BlockSpec((_BN, 2 * h), lambda i: (i, 0)),
            pl.BlockSpec((_BN, d), lambda i: (i, 0)),
            pl.BlockSpec((1, 1, _BN), lambda i: (i, 0, 0)),
            pl.BlockSpec((d + h, 128), lambda i: (0, 0)),
            pl.BlockSpec((1, 128), lambda i: (0, 0)),
            pl.BlockSpec((128, 1), lambda i: (0, 0)),
            pl.BlockSpec((1, 1), lambda i: (0, 0)),
        ],
        out_specs=pl.BlockSpec((_G, 1), lambda i: (0, 0)),
        out_shape=jax.ShapeDtypeStruct((_G, 1), jnp.float32),
        scratch_shapes=[pltpu.VMEM((_G, 384), jnp.float32)],
    )(o[:_NP], o[_NP:], x, batch_p,
      V1, c1.reshape(1, 128), V2, c2.reshape(1, 1))
    return y
